# baseline (speedup 1.0000x reference)
import jax
import jax.numpy as jnp
from jax.experimental import pallas as pl

BN_EPS = 1e-5


def _gcn_conv(x, edge_index, W, b, edge_weight=None):
    N = x.shape[0]
    src = edge_index[0]
    dst = edge_index[1]
    if edge_weight is None:
        ew = jnp.ones(src.shape[0], x.dtype)
    else:
        ew = edge_weight
    loop = jnp.arange(N, dtype=src.dtype)
    src = jnp.concatenate([src, loop])
    dst = jnp.concatenate([dst, loop])
    ew = jnp.concatenate([ew, jnp.ones(N, x.dtype)])
    deg = jnp.zeros((N,), x.dtype).at[dst].add(ew)
    dinv = jnp.where(deg > 0, 1.0 / jnp.sqrt(deg), 0.0)
    norm = dinv[src] * ew * dinv[dst]
    xw = x @ W
    out = jnp.zeros((N, W.shape[1]), x.dtype).at[dst].add(xw[src] * norm[:, None])
    return out + b


def _gcn_branch(x, ei, convs, bns, ew, nodenum):
    n = len(convs)
    for i in range(n):
        W, b = convs[i]
        x = _gcn_conv(x, ei, W, b, ew)
        if i == n - 1:
            nb = x.shape[0] // nodenum
            x = x.reshape(nb, nodenum, x.shape[1]).mean(axis=1)
            x = jnp.tanh(x)
        else:
            g, beta = bns[i]
            x = g * x / jnp.sqrt(1.0 + BN_EPS) + beta
            x = jax.nn.elu(x)
    return x


def _mlp(x, layers):
    n = len(layers)
    for i in range(n):
        W, b = layers[i]
        x = x @ W + b
        if i != n - 1:
            x = jax.nn.relu(x)
    return x


def _id_kernel(x_ref, o_ref):
    o_ref[...] = x_ref[...]


def kernel(Drug1_F, Drug2_F, Drug1_ADJ, Drug2_ADJ, EXP1, EXP2, EXP_ADJ, EXP_ADJ_WGT, syn, params):
    bsz = syn.shape[0]
    nn_chem = Drug1_F.shape[0] // bsz
    nn_exp = EXP1.shape[0] // bsz
    d1 = _gcn_branch(Drug1_F, Drug1_ADJ, params["chem1"], params["bn_chem1"], None, nn_chem)
    d2 = _gcn_branch(Drug2_F, Drug2_ADJ, params["chem2"], params["bn_chem2"], None, nn_chem)
    e1 = _gcn_branch(EXP1, EXP_ADJ, params["exp1"], params["bn_exp1"], EXP_ADJ_WGT, nn_exp)
    e2 = _gcn_branch(EXP2, EXP_ADJ, params["exp2"], params["bn_exp2"], EXP_ADJ_WGT, nn_exp)
    in1 = jnp.concatenate([d1, e1], axis=1)
    in2 = jnp.concatenate([d2, e2], axis=1)
    h1 = _mlp(in1, params["mlp1"])
    h2 = _mlp(in2, params["mlp2"])
    X = jnp.concatenate([h1, h2], axis=1)
    out = _mlp(X, params["snp"])
    return pl.pallas_call(
        _id_kernel,
        out_shape=jax.ShapeDtypeStruct(out.shape, out.dtype),
    )(out)


# speedup vs baseline: 11.9936x; 11.9936x over previous
import functools

import jax
import jax.numpy as jnp
from jax import lax
from jax.experimental import pallas as pl
from jax.experimental.pallas import tpu as pltpu
from jax.experimental.pallas import tpu_sc as plsc

BN_EPS = 1e-5
_BN_SCALE = 1.0 / (1.0 + BN_EPS) ** 0.5


def _sc_degree(dst, ew, N, E, T):
    use_ones = ew is None
    epw = E // 16
    nch = epw // T
    zeros = jnp.zeros((N, 16), jnp.float32)
    ins = (dst, zeros) if use_ones else (dst, ew, zeros)

    mesh = plsc.VectorSubcoreMesh(core_axis_name="c", subcore_axis_name="s")
    scratch = [
        pltpu.VMEM((T,), jnp.int32),
        pltpu.VMEM((T,), jnp.float32),
        pltpu.VMEM((T, 16), jnp.float32),
        pltpu.VMEM_SHARED((N, 16), jnp.float32),
    ]

    def body(*refs):
        if use_ones:
            dst_hbm, zeros_hbm, out_hbm, dst_v, ew_v, rows_v, acc = refs
            ew_hbm = None
        else:
            dst_hbm, ew_hbm, zeros_hbm, out_hbm, dst_v, ew_v, rows_v, acc = refs
        cid = lax.axis_index("c")
        sid = lax.axis_index("s")

        @pl.when(cid == 0)
        def _():
            rp = N // 16
            pltpu.sync_copy(zeros_hbm.at[pl.ds(sid * rp, rp)],
                            acc.at[pl.ds(sid * rp, rp)])
            if use_ones:
                ones16 = jnp.ones((16,), jnp.float32)

                def fill(i, c):
                    rows_v[i] = ones16
                    return c

                lax.fori_loop(0, T, fill, 0)
            plsc.subcore_barrier()
            base = sid * epw

            def step(t, c):
                off = base + t * T
                pltpu.sync_copy(dst_hbm.at[pl.ds(off, T)], dst_v)
                if not use_ones:
                    pltpu.sync_copy(ew_hbm.at[pl.ds(off, T)], ew_v)

                    def group(g, cc):
                        w16 = ew_v[pl.ds(g * 16, 16)]
                        for j in range(16):
                            rows_v[g * 16 + j] = w16[j] * jnp.ones(
                                (16,), jnp.float32)
                        return cc

                    lax.fori_loop(0, T // 16, group, 0)
                pltpu.sync_copy(rows_v, acc.at[dst_v], add=True)
                return c

            lax.fori_loop(0, nch, step, 0)
            plsc.subcore_barrier()
            pltpu.sync_copy(acc.at[pl.ds(sid * rp, rp)],
                            out_hbm.at[pl.ds(sid * rp, rp)])

    return pl.kernel(
        body,
        out_type=jax.ShapeDtypeStruct((N, 16), jnp.float32),
        mesh=mesh,
        compiler_params=pltpu.CompilerParams(use_tc_tiling_on_sc=False),
        scratch_types=scratch,
    )(*ins)


def _sc_agg(yL, yR, src, dst, ew, N, E, Hh, T):
    epw = E // 16
    nch = epw // T
    has_ew = ew is not None
    ins = (yL, yR, src, dst) + ((ew,) if has_ew else ())

    mesh = plsc.VectorSubcoreMesh(core_axis_name="c", subcore_axis_name="s")
    scratch = [
        pltpu.VMEM((T,), jnp.int32),
        pltpu.VMEM((T,), jnp.int32),
        pltpu.VMEM((T,), jnp.float32),
        pltpu.VMEM((T, Hh), jnp.float32),
        pltpu.VMEM_SHARED((N, Hh), jnp.float32),
        pltpu.SemaphoreType.DMA,
    ]

    def body(*refs):
        yL_hbm, yR_hbm, src_hbm, dst_hbm = refs[:4]
        rest = refs[4:]
        if has_ew:
            ew_hbm, rest = rest[0], rest[1:]
        outL, outR, src_v, dst_v, ew_v, rows_v, acc, sem = rest
        cid = lax.axis_index("c")
        sid = lax.axis_index("s")
        rp = N // 16

        def run(y_hbm, out_hbm):
            pltpu.sync_copy(y_hbm.at[pl.ds(sid * rp, rp)],
                            acc.at[pl.ds(sid * rp, rp)])
            plsc.subcore_barrier()
            base = sid * epw

            def step(t, c):
                off = base + t * T
                pltpu.sync_copy(src_hbm.at[pl.ds(off, T)], src_v)
                pltpu.sync_copy(dst_hbm.at[pl.ds(off, T)], dst_v)
                if has_ew:
                    pltpu.sync_copy(ew_hbm.at[pl.ds(off, T)], ew_v)
                pltpu.async_copy(y_hbm.at[src_v], rows_v, sem).wait()
                if has_ew:
                    def scale(g, cc):
                        w16 = ew_v[pl.ds(g * 16, 16)]
                        for j in range(16):
                            e = g * 16 + j
                            rows_v[e] = rows_v[e] * w16[j]
                        return cc
                    lax.fori_loop(0, T // 16, scale, 0)
                pltpu.sync_copy(rows_v, acc.at[dst_v], add=True)
                return c

            lax.fori_loop(0, nch, step, 0)
            plsc.subcore_barrier()
            pltpu.sync_copy(acc.at[pl.ds(sid * rp, rp)],
                            out_hbm.at[pl.ds(sid * rp, rp)])

        @pl.when(cid == 0)
        def _():
            run(yL_hbm, outL)

        @pl.when(cid == 1)
        def _():
            run(yR_hbm, outR)

    st = jax.ShapeDtypeStruct((N, Hh), jnp.float32)
    return pl.kernel(
        body,
        out_type=(st, st),
        mesh=mesh,
        compiler_params=pltpu.CompilerParams(use_tc_tiling_on_sc=False),
        scratch_types=scratch,
    )(*ins)


def _dinv_kernel(deg16):
    N = deg16.shape[0]

    def kern(d_ref, o_ref):
        o_ref[...] = lax.rsqrt(d_ref[...][:, :1] + 1.0)

    BN = 2000
    return pl.pallas_call(
        kern,
        grid=(N // BN,),
        in_specs=[pl.BlockSpec((BN, 16), lambda i: (i, 0))],
        out_specs=pl.BlockSpec((BN, 1), lambda i: (i, 0)),
        out_shape=jax.ShapeDtypeStruct((N, 1), jnp.float32),
    )(deg16)


def _l1_kernel(x, W, dinv, Hh, BN):
    N, D = x.shape
    H = W.shape[1]

    def kern(x_ref, w_ref, dv_ref, oL, oR):
        xw = jnp.dot(x_ref[...], w_ref[...],
                     preferred_element_type=jnp.float32) * dv_ref[...]
        oL[...] = xw[:, :Hh]
        oR[...] = xw[:, Hh:]

    st = jax.ShapeDtypeStruct((N, Hh), jnp.float32)
    return pl.pallas_call(
        kern,
        grid=(N // BN,),
        in_specs=[
            pl.BlockSpec((BN, D), lambda i: (i, 0)),
            pl.BlockSpec((D, H), lambda i: (0, 0)),
            pl.BlockSpec((BN, 1), lambda i: (i, 0)),
        ],
        out_specs=[pl.BlockSpec((BN, Hh), lambda i: (i, 0))] * 2,
        out_shape=[st, st],
    )(x, W, dinv)


def _mid_kernel(aL, aR, dinv, b, g, beta, W2, Hh, BN):
    N = aL.shape[0]
    H = 2 * Hh

    def kern(aL_ref, aR_ref, dv_ref, b_ref, g_ref, be_ref, w_ref, oL, oR):
        dv = dv_ref[...]
        h = jnp.concatenate([aL_ref[...], aR_ref[...]], axis=1) * dv + b_ref[...]
        h = g_ref[...] * h * _BN_SCALE + be_ref[...]
        h = jnp.where(h > 0, h, jnp.exp(h) - 1.0)
        xw = jnp.dot(h, w_ref[...], preferred_element_type=jnp.float32) * dv
        oL[...] = xw[:, :Hh]
        oR[...] = xw[:, Hh:]

    st = jax.ShapeDtypeStruct((N, Hh), jnp.float32)
    return pl.pallas_call(
        kern,
        grid=(N // BN,),
        in_specs=[
            pl.BlockSpec((BN, Hh), lambda i: (i, 0)),
            pl.BlockSpec((BN, Hh), lambda i: (i, 0)),
            pl.BlockSpec((BN, 1), lambda i: (i, 0)),
            pl.BlockSpec((1, H), lambda i: (0, 0)),
            pl.BlockSpec((1, H), lambda i: (0, 0)),
            pl.BlockSpec((1, H), lambda i: (0, 0)),
            pl.BlockSpec((H, H), lambda i: (0, 0)),
        ],
        out_specs=[pl.BlockSpec((BN, Hh), lambda i: (i, 0))] * 2,
        out_shape=[st, st],
    )(aL, aR, dinv, b, g, beta, W2)


def _pool_kernel(aL, aR, dinv, b, nodenum, nb, G):
    Hh = aL.shape[1]
    H = 2 * Hh
    R = G * nodenum

    def kern(aL_ref, aR_ref, dv_ref, b_ref, o_hbm, acc_v, sem):
        h = (jnp.concatenate([aL_ref[...], aR_ref[...]], axis=1) * dv_ref[...]
             + b_ref[...])
        row = lax.broadcasted_iota(jnp.int32, (G, R), 0)
        col = lax.broadcasted_iota(jnp.int32, (G, R), 1)
        pm = jnp.where(col // nodenum == row, 1.0 / nodenum, 0.0)
        acc_v[...] = jnp.tanh(
            jnp.dot(pm, h, preferred_element_type=jnp.float32))
        i = pl.program_id(0)
        cp = pltpu.make_async_copy(acc_v, o_hbm.at[pl.ds(i * G, G)], sem)
        cp.start()
        cp.wait()

    return pl.pallas_call(
        kern,
        grid=(nb // G,),
        in_specs=[
            pl.BlockSpec((R, Hh), lambda i: (i, 0)),
            pl.BlockSpec((R, Hh), lambda i: (i, 0)),
            pl.BlockSpec((R, 1), lambda i: (i, 0)),
            pl.BlockSpec((1, H), lambda i: (0, 0)),
        ],
        out_specs=pl.BlockSpec(memory_space=pl.ANY),
        out_shape=jax.ShapeDtypeStruct((nb, H), jnp.float32),
        scratch_shapes=[pltpu.VMEM((G, H), jnp.float32),
                        pltpu.SemaphoreType.DMA],
    )(aL, aR, dinv, b)


def _head_kernel(d1, e1, d2, e2, p):
    nb = d1.shape[0]
    w1a, b1a = p["mlp1"][0]
    w1b, b1b = p["mlp1"][1]
    w2a, b2a = p["mlp2"][0]
    w2b, b2b = p["mlp2"][1]
    ws1, bs1 = p["snp"][0]
    ws2, bs2 = p["snp"][1]
    ws3, bs3 = p["snp"][2]

    def kern(d1r, e1r, d2r, e2r, w1ar, b1ar, w1br, b1br, w2ar, b2ar, w2br,
             b2br, ws1r, bs1r, ws2r, bs2r, ws3r, bs3r, o_ref):
        def mm(x, w, bb):
            return jnp.dot(x, w[...], preferred_element_type=jnp.float32) + bb[...]

        in1 = jnp.concatenate([d1r[...], e1r[...]], axis=1)
        in2 = jnp.concatenate([d2r[...], e2r[...]], axis=1)
        h1 = mm(jnp.maximum(mm(in1, w1ar, b1ar), 0.0), w1br, b1br)
        h2 = mm(jnp.maximum(mm(in2, w2ar, b2ar), 0.0), w2br, b2br)
        X = jnp.concatenate([h1, h2], axis=1)
        X = jnp.maximum(mm(X, ws1r, bs1r), 0.0)
        X = jnp.maximum(mm(X, ws2r, bs2r), 0.0)
        o_ref[...] = mm(X, ws3r, bs3r)

    args = (d1, e1, d2, e2,
            w1a, b1a.reshape(1, -1), w1b, b1b.reshape(1, -1),
            w2a, b2a.reshape(1, -1), w2b, b2b.reshape(1, -1),
            ws1, bs1.reshape(1, -1), ws2, bs2.reshape(1, -1),
            ws3, bs3.reshape(1, -1))
    return pl.pallas_call(
        kern,
        out_shape=jax.ShapeDtypeStruct((nb, 1), jnp.float32),
    )(*args)


def _branch(x, src, dst, ew, dinv, convs, bns, nodenum, nb, Hh, BN, T_sc):
    N = x.shape[0]
    E = src.shape[0]
    (W1, b1), (W2, b2), (W3, b3) = convs
    (g1, be1), (g2, be2) = bns
    r = lambda v: v.reshape(1, -1)

    yL, yR = _l1_kernel(x, W1, dinv, Hh, BN)
    aL, aR = _sc_agg(yL, yR, src, dst, ew, N, E, Hh, T_sc)
    yL, yR = _mid_kernel(aL, aR, dinv, r(b1), r(g1), r(be1), W2, Hh, BN)
    aL, aR = _sc_agg(yL, yR, src, dst, ew, N, E, Hh, T_sc)
    yL, yR = _mid_kernel(aL, aR, dinv, r(b2), r(g2), r(be2), W3, Hh, BN)
    aL, aR = _sc_agg(yL, yR, src, dst, ew, N, E, Hh, T_sc)
    G = 4 if (nodenum % 8) else 1
    return _pool_kernel(aL, aR, dinv, r(b3), nodenum, nb, G)


def kernel(Drug1_F, Drug2_F, Drug1_ADJ, Drug2_ADJ, EXP1, EXP2, EXP_ADJ,
           EXP_ADJ_WGT, syn, params):
    nb = syn.shape[0]
    n_chem = Drug1_F.shape[0]
    n_exp = EXP1.shape[0]
    nn_chem = n_chem // nb
    nn_exp = n_exp // nb
    BN = 2000

    s1, t1 = Drug1_ADJ[0], Drug1_ADJ[1]
    s2, t2 = Drug2_ADJ[0], Drug2_ADJ[1]
    se, te = EXP_ADJ[0], EXP_ADJ[1]

    dinv1 = _dinv_kernel(_sc_degree(t1, None, n_chem, s1.shape[0], 2000))
    dinv2 = _dinv_kernel(_sc_degree(t2, None, n_chem, s2.shape[0], 2000))
    dinve = _dinv_kernel(_sc_degree(te, EXP_ADJ_WGT, n_exp, se.shape[0], 800))

    d1 = _branch(Drug1_F, s1, t1, None, dinv1, params["chem1"],
                 params["bn_chem1"], nn_chem, nb, 32, BN, 200)
    d2 = _branch(Drug2_F, s2, t2, None, dinv2, params["chem2"],
                 params["bn_chem2"], nn_chem, nb, 32, BN, 200)
    e1 = _branch(EXP1, se, te, EXP_ADJ_WGT, dinve, params["exp1"],
                 params["bn_exp1"], nn_exp, nb, 16, BN, 800)
    e2 = _branch(EXP2, se, te, EXP_ADJ_WGT, dinve, params["exp2"],
                 params["bn_exp2"], nn_exp, nb, 16, BN, 800)

    return _head_kernel(d1, e1, d2, e2, params)
